# Initial kernel scaffold; baseline (speedup 1.0000x reference)
import jax
import jax.numpy as jnp
from jax.experimental import pallas as pl


def kernel(q, e1, e2):
    raise NotImplementedError("write your pallas kernel here")


# speedup vs baseline: 41.8942x; 41.8942x over previous
import jax
import jax.numpy as jnp
from jax import lax
from jax.experimental import pallas as pl
from jax.experimental.pallas import tpu as pltpu
from jax.experimental.pallas import tpu_sc as plsc

NUM_HEADS = 16
SEQ_LEN = 2048
TBL_W = 4088

_NC = 2
_NS = 16
_NW = _NC * _NS
_ROWS_PER_W = NUM_HEADS * SEQ_LEN // _NW
_GROUP = 8


def _sc_body(table_hbm, out_hbm, table_v, sem):
    c = lax.axis_index("c")
    s = lax.axis_index("s")
    wid = s * _NC + c
    h = wid // 2
    half = wid % 2
    src = pl.multiple_of(h * (8 * TBL_W), 8)
    pltpu.sync_copy(table_hbm.at[pl.ds(src, 8 * TBL_W)], table_v)
    i0 = half * _ROWS_PER_W

    def group(g, carry):
        ibase = i0 + g * _GROUP
        copies = []
        for b in range(_GROUP):
            i = ibase + b
            start = (SEQ_LEN - 1) - i
            a = start & 7
            off = pl.multiple_of(a * TBL_W + (start - a), 8)
            dst = pl.multiple_of((h * SEQ_LEN + i) * SEQ_LEN, 8)
            copies.append(pltpu.make_async_copy(
                table_v.at[pl.ds(off, SEQ_LEN)],
                out_hbm.at[pl.ds(dst, SEQ_LEN)], sem))
        for cp in copies:
            cp.start()
        for cp in copies:
            cp.wait()
        return carry

    lax.fori_loop(0, _ROWS_PER_W // _GROUP, group, 0)


def kernel(q, e1, e2):
    heads = e1.shape[1]
    seq = e1.shape[0]
    c_rev = jnp.concatenate([e1[::-1], e2], axis=0)
    table = jnp.transpose(c_rev)
    rep = jnp.stack([table[:, a:a + TBL_W] for a in range(8)], axis=1)
    rep = rep.reshape(heads * 8 * TBL_W)

    mesh = plsc.VectorSubcoreMesh(core_axis_name="c", subcore_axis_name="s")
    out = pl.kernel(
        _sc_body,
        out_type=jax.ShapeDtypeStruct((heads * seq * seq,), jnp.float32),
        mesh=mesh,
        scratch_types=[
            pltpu.VMEM((8 * TBL_W,), jnp.float32),
            pltpu.SemaphoreType.DMA,
        ],
    )(rep)
    out = out.reshape(heads, seq, seq)

    batch_dim = q.shape[0] // heads
    if batch_dim != 1:
        out = jnp.tile(out, (batch_dim, 1, 1))
    return out


# speedup vs baseline: 42.1155x; 1.0053x over previous
import jax
import jax.numpy as jnp
from jax import lax
from jax.experimental import pallas as pl
from jax.experimental.pallas import tpu as pltpu
from jax.experimental.pallas import tpu_sc as plsc

NUM_HEADS = 16
SEQ_LEN = 2048
TBL_W = 4088

_NC = 2
_NS = 16
_NW = _NC * _NS
_ROWS_PER_W = NUM_HEADS * SEQ_LEN // _NW
_GROUP = 8


def _sc_body(table_hbm, out_hbm, table_v, sem):
    c = lax.axis_index("c")
    s = lax.axis_index("s")
    wid = s * _NC + c
    h = wid // 2
    half = wid % 2
    src = pl.multiple_of(h * (8 * TBL_W), 8)
    pltpu.sync_copy(table_hbm.at[pl.ds(src, 8 * TBL_W)], table_v)
    i0 = half * _ROWS_PER_W

    def fire(g):
        ibase = i0 + g * _GROUP
        for b in range(_GROUP):
            i = ibase + b
            start = (SEQ_LEN - 1) - i
            a = start & 7
            off = pl.multiple_of(a * TBL_W + (start - a), 8)
            dst = pl.multiple_of((h * SEQ_LEN + i) * SEQ_LEN, 8)
            pltpu.make_async_copy(
                table_v.at[pl.ds(off, SEQ_LEN)],
                out_hbm.at[pl.ds(dst, SEQ_LEN)], sem).start()

    def drain():
        for b in range(_GROUP):
            pltpu.make_async_copy(
                table_hbm.at[pl.ds(0, SEQ_LEN)],
                table_v.at[pl.ds(0, SEQ_LEN)], sem).wait()

    def group(g, carry):
        fire(g)

        @pl.when(g > 0)
        def _():
            drain()

        return carry

    lax.fori_loop(0, _ROWS_PER_W // _GROUP, group, 0)
    drain()


def kernel(q, e1, e2):
    heads = e1.shape[1]
    seq = e1.shape[0]
    c_rev = jnp.concatenate([e1[::-1], e2], axis=0)
    table = jnp.transpose(c_rev)
    rep = jnp.stack([table[:, a:a + TBL_W] for a in range(8)], axis=1)
    rep = rep.reshape(heads * 8 * TBL_W)

    mesh = plsc.VectorSubcoreMesh(core_axis_name="c", subcore_axis_name="s")
    out = pl.kernel(
        _sc_body,
        out_type=jax.ShapeDtypeStruct((heads * seq * seq,), jnp.float32),
        mesh=mesh,
        scratch_types=[
            pltpu.VMEM((8 * TBL_W,), jnp.float32),
            pltpu.SemaphoreType.DMA,
        ],
    )(rep)
    out = out.reshape(heads, seq, seq)

    batch_dim = q.shape[0] // heads
    if batch_dim != 1:
        out = jnp.tile(out, (batch_dim, 1, 1))
    return out


# speedup vs baseline: 98.2379x; 2.3326x over previous
import jax
import jax.numpy as jnp
from jax import lax
from jax.experimental import pallas as pl
from jax.experimental.pallas import tpu as pltpu
from jax.experimental.pallas import tpu_sc as plsc

NUM_HEADS = 16
SEQ_LEN = 2048
TBL_W = 4096
_NC = 2
_NS = 16
_GROUPS = SEQ_LEN // 8
_NTILE = 31
_CLS_PER_W = 8


def _sc_body(table_hbm, out_hbm, table_v, stg_v, sem):
    c = lax.axis_index("c")
    s = lax.axis_index("s")
    wid = s * _NC + c
    h = wid // 2
    cls0 = (wid % 2) * _CLS_PER_W
    src = pl.multiple_of(h * TBL_W, 8)
    pltpu.sync_copy(table_hbm.at[pl.ds(src, TBL_W)], table_v)
    iota = lax.iota(jnp.int32, 16)

    for k in range(_CLS_PER_W):
        cls = cls0 + k

        def stage_m(m, carry, cls=cls):
            base = 127 + 128 * m - 8 * cls
            for r in range(8):
                for kk in range(8):
                    v = table_v[pl.ds(base - r + 16 * kk, 16)]
                    stg_v[m, r, pl.ds(16 * kk, 16)] = v
            return carry

        lax.fori_loop(0, _NTILE, stage_m, 0)

        cps = []
        for j in range(16):
            g = cls + 16 * j
            cps.append(pltpu.make_async_copy(
                stg_v.at[pl.ds(15 - j, 16)], out_hbm.at[h, g], sem))
        for cp in cps:
            cp.start()
        for cp in cps:
            cp.wait()


def kernel(q, e1, e2):
    heads = e1.shape[1]
    seq = e1.shape[0]
    c_rev = jnp.concatenate([e1[::-1], e2], axis=0)
    table = jnp.transpose(c_rev)
    table = jnp.pad(table, ((0, 0), (0, TBL_W - (2 * seq - 1))))
    table = table.reshape(heads * TBL_W)

    mesh = plsc.VectorSubcoreMesh(core_axis_name="c", subcore_axis_name="s")
    out5 = pl.kernel(
        _sc_body,
        out_type=jax.ShapeDtypeStruct(
            (heads, seq // 8, seq // 128, 8, 128), jnp.float32),
        mesh=mesh,
        scratch_types=[
            pltpu.VMEM((TBL_W,), jnp.float32),
            pltpu.VMEM((_NTILE, 8, 128), jnp.float32),
            pltpu.SemaphoreType.DMA,
        ],
    )(table)
    out = out5.transpose(0, 1, 3, 2, 4).reshape(heads, seq, seq)

    batch_dim = q.shape[0] // heads
    if batch_dim != 1:
        out = jnp.tile(out, (batch_dim, 1, 1))
    return out


# speedup vs baseline: 136.7377x; 1.3919x over previous
import jax
import jax.numpy as jnp
from jax import lax
from jax.experimental import pallas as pl
from jax.experimental.pallas import tpu as pltpu
from jax.experimental.pallas import tpu_sc as plsc

NUM_HEADS = 16
SEQ_LEN = 2048
TBL_W = 4096
_NC = 2
_NS = 16
_GROUPS = SEQ_LEN // 8
_NTILE = 31
_CLS_PER_W = 8


def _sc_body(table_hbm, out_hbm, table_v, stg_v, sem):
    c = lax.axis_index("c")
    s = lax.axis_index("s")
    wid = s * _NC + c
    h = wid // 2
    cls0 = (wid % 2) * _CLS_PER_W
    src = pl.multiple_of(h * TBL_W, 8)
    pltpu.sync_copy(table_hbm.at[pl.ds(src, TBL_W)], table_v)
    iota = lax.iota(jnp.int32, 16)

    def drain16():
        for _ in range(16):
            pltpu.make_async_copy(
                out_hbm.at[h, 0], stg_v.at[0, pl.ds(0, 16)], sem).wait()

    for k in range(_CLS_PER_W):
        cls = cls0 + k
        buf = k % 2

        if k >= 2:
            drain16()

        def stage_m(m, carry, cls=cls, buf=buf):
            base = 127 + 128 * m - 8 * cls
            for r in range(8):
                for kk in range(8):
                    v = table_v[pl.ds(base - r + 16 * kk, 16)]
                    stg_v[buf, m, r, pl.ds(16 * kk, 16)] = v
            return carry

        lax.fori_loop(0, _NTILE, stage_m, 0)

        for j in range(16):
            g = cls + 16 * j
            pltpu.make_async_copy(
                stg_v.at[buf, pl.ds(15 - j, 16)], out_hbm.at[h, g],
                sem).start()

    drain16()
    drain16()


def kernel(q, e1, e2):
    heads = e1.shape[1]
    seq = e1.shape[0]
    c_rev = jnp.concatenate([e1[::-1], e2], axis=0)
    table = jnp.transpose(c_rev)
    table = jnp.pad(table, ((0, 0), (0, TBL_W - (2 * seq - 1))))
    table = table.reshape(heads * TBL_W)

    mesh = plsc.VectorSubcoreMesh(core_axis_name="c", subcore_axis_name="s")
    out5 = pl.kernel(
        _sc_body,
        out_type=jax.ShapeDtypeStruct(
            (heads, seq // 8, seq // 128, 8, 128), jnp.float32),
        mesh=mesh,
        scratch_types=[
            pltpu.VMEM((TBL_W,), jnp.float32),
            pltpu.VMEM((2, _NTILE, 8, 128), jnp.float32),
            pltpu.SemaphoreType.DMA,
        ],
    )(table)
    out = out5.transpose(0, 1, 3, 2, 4).reshape(heads, seq, seq)

    batch_dim = q.shape[0] // heads
    if batch_dim != 1:
        out = jnp.tile(out, (batch_dim, 1, 1))
    return out
